# baseline (speedup 1.0000x reference)
import functools

import jax
import jax.numpy as jnp
from jax import lax
from jax.experimental import pallas as pl
from jax.experimental.pallas import tpu as pltpu
from jax.experimental.pallas import tpu_sc as plsc

VOCAB = 1_000_000
DIM = 64
DIMP = 128
BATCH = 4096
CTX = 20
NEG = 10

NPOS = CTX
NPOSP = 24
NNEG = CTX * NEG
ROWS_B = NPOSP + NNEG
LANES = 16
GROUPS_B = ROWS_B // LANES

NWORKERS = 32
B_PER_W = BATCH // NWORKERS
NBUF = 4
CHUNK_ROWS = ROWS_B
N_CHUNKS = B_PER_W
GATHERS = 2
GLEN = CHUNK_ROWS // GATHERS
TOTAL_ROWS = BATCH * ROWS_B
TC_COLS = 128
TC_ROWS = TOTAL_ROWS // TC_COLS
TC_GRID = 8
TC_BLK = TC_ROWS // TC_GRID


def _sc_inrows_body(inword_hbm, winp_hbm, out_hbm, inidx_v, inrows_v, sem):
    nc = 2
    wid = lax.axis_index("s") * nc + lax.axis_index("c")
    pltpu.sync_copy(inword_hbm.at[pl.ds(wid * B_PER_W, B_PER_W)], inidx_v)
    pltpu.async_copy(winp_hbm.at[inidx_v], inrows_v, sem).wait()
    pltpu.sync_copy(inrows_v, out_hbm.at[pl.ds(wid * B_PER_W, B_PER_W)])


def _sc_inrows(inword, W_in_p):
    mesh = plsc.VectorSubcoreMesh(core_axis_name="c", subcore_axis_name="s")
    k = functools.partial(
        pl.kernel,
        mesh=mesh,
        out_type=jax.ShapeDtypeStruct((BATCH, DIMP), jnp.float32),
        compiler_params=pltpu.CompilerParams(
            needs_layout_passes=False, use_tc_tiling_on_sc=False),
        scratch_types=[
            pltpu.VMEM((B_PER_W,), jnp.int32),
            pltpu.VMEM((B_PER_W, DIMP), jnp.float32),
            pltpu.SemaphoreType.DMA,
        ],
    )(_sc_inrows_body)
    return k(inword, W_in_p)


def _sc_body(irows_hbm, owp_hbm, nw_hbm, wout_hbm, out_hbm,
             inrows_v, cidx_v, rows_v, sc_v, *sems):
    nc = 2
    wid = lax.axis_index("s") * nc + lax.axis_index("c")
    sem_row = sems[0:NBUF]
    sem_idx = sems[NBUF:2 * NBUF]
    sem_sc = sems[2 * NBUF:3 * NBUF]

    pltpu.sync_copy(irows_hbm.at[pl.ds(wid * B_PER_W, B_PER_W)], inrows_v)

    iota = lax.iota(jnp.int32, LANES)
    base_b = wid * B_PER_W

    def fire_idx(chunk, buf):
        b = base_b + chunk
        pltpu.async_copy(
            owp_hbm.at[b], cidx_v.at[buf, pl.ds(0, NPOSP)], sem_idx[buf])
        pltpu.async_copy(
            nw_hbm.at[b], cidx_v.at[buf, pl.ds(NPOSP, NNEG)], sem_idx[buf])

    def drain_idx(buf):
        pltpu.make_async_copy(
            owp_hbm.at[0], cidx_v.at[buf, pl.ds(0, NPOSP)],
            sem_idx[buf]).wait()
        pltpu.make_async_copy(
            nw_hbm.at[0], cidx_v.at[buf, pl.ds(NPOSP, NNEG)],
            sem_idx[buf]).wait()

    def fire_rows(buf):
        for j in range(GATHERS):
            pltpu.async_copy(
                wout_hbm.at[cidx_v.at[buf, pl.ds(j * GLEN, GLEN)]],
                rows_v.at[buf, pl.ds(j * GLEN, GLEN)],
                sem_row[buf])

    def drain_rows(buf):
        for j in range(GATHERS):
            pltpu.make_async_copy(
                wout_hbm.at[pl.ds(0, GLEN)],
                rows_v.at[buf, pl.ds(j * GLEN, GLEN)],
                sem_row[buf]).wait()

    for b in range(NBUF):
        fire_idx(b, b)
    for b in range(NBUF - 1):
        drain_idx(b)
        fire_rows(b)

    def quad_body(cc, carry):
        for par in range(NBUF):
            chunk = cc * NBUF + par
            buf = par

            drain_rows(buf)

            @pl.when(chunk + NBUF - 1 < N_CHUNKS)
            def _fire():
                drain_idx((buf + NBUF - 1) % NBUF)
                fire_rows((buf + NBUF - 1) % NBUF)

            @pl.when(chunk + NBUF < N_CHUNKS)
            def _pref():
                fire_idx(chunk + NBUF, buf)

            @pl.when(chunk >= NBUF)
            def _drain_sc():
                pltpu.make_async_copy(
                    out_hbm.at[pl.ds(0, CHUNK_ROWS)], sc_v.at[buf],
                    sem_sc[buf]).wait()

            b_idx = chunk
            wv = [inrows_v[b_idx, pl.ds(q * LANES, LANES)]
                  for q in range(DIM // LANES)]

            def group_body(g, gc, wv=wv, buf=buf):
                base_row = g * LANES
                res = jnp.zeros((LANES,), jnp.float32)
                for r in range(LANES):
                    row = base_row + r
                    p = ((rows_v[buf, row, pl.ds(0, LANES)] * wv[0]
                          + rows_v[buf, row, pl.ds(LANES, LANES)] * wv[1])
                         + (rows_v[buf, row, pl.ds(2 * LANES, LANES)] * wv[2]
                            + rows_v[buf, row, pl.ds(3 * LANES, LANES)]
                            * wv[3]))
                    s = jnp.sum(p)
                    res = jnp.where(iota == r, s, res)
                sc_v[buf, pl.ds(base_row, LANES)] = res
                return gc

            lax.fori_loop(0, GROUPS_B, group_body, 0)

            gchunk = wid * N_CHUNKS + chunk
            pltpu.async_copy(
                sc_v.at[buf],
                out_hbm.at[pl.ds(gchunk * CHUNK_ROWS, CHUNK_ROWS)],
                sem_sc[buf])
        return carry

    lax.fori_loop(0, N_CHUNKS // NBUF, quad_body, 0)
    for b in range(NBUF):
        pltpu.make_async_copy(
            out_hbm.at[pl.ds(0, CHUNK_ROWS)], sc_v.at[b],
            sem_sc[b]).wait()


def _sc_scores(inrows, owp, negword, W_out):
    mesh = plsc.VectorSubcoreMesh(core_axis_name="c", subcore_axis_name="s")
    k = functools.partial(
        pl.kernel,
        mesh=mesh,
        out_type=jax.ShapeDtypeStruct((TOTAL_ROWS,), jnp.float32),
        compiler_params=pltpu.CompilerParams(
            needs_layout_passes=False, use_tc_tiling_on_sc=False),
        scratch_types=[
            pltpu.VMEM((B_PER_W, DIMP), jnp.float32),
            pltpu.VMEM((NBUF, CHUNK_ROWS), jnp.int32),
            pltpu.VMEM((NBUF, CHUNK_ROWS, DIM), jnp.float32),
            pltpu.VMEM((NBUF, CHUNK_ROWS), jnp.float32),
        ] + [pltpu.SemaphoreType.DMA] * (3 * NBUF),
    )(_sc_body)
    return k(inrows, owp, negword, W_out)


def _tc_loss_body(s_ref, o_ref):
    pid = pl.program_id(0)
    x = s_ref[...]
    r = lax.broadcasted_iota(jnp.int32, (TC_BLK, TC_COLS), 0)
    c = lax.broadcasted_iota(jnp.int32, (TC_BLK, TC_COLS), 1)
    flat = (pid * TC_BLK + r) * TC_COLS + c
    j = flat % ROWS_B
    z = jnp.where(j < NPOS, x, -x)
    ls = jnp.minimum(z, 0.0) - jnp.log(1.0 + jnp.exp(-jnp.abs(z)))
    pad = jnp.logical_and(j >= NPOS, j < NPOSP)
    val = jnp.where(pad, 0.0, ls)

    @pl.when(pid == 0)
    def _init():
        o_ref[0, 0] = 0.0

    o_ref[0, 0] += jnp.sum(val)

    @pl.when(pid == TC_GRID - 1)
    def _fini():
        o_ref[0, 0] = o_ref[0, 0] * (-1.0 / (BATCH * CTX))


def kernel(inword, outword, negword, W_in, W_out):
    owp = jnp.concatenate(
        [outword, jnp.zeros((BATCH, NPOSP - NPOS), jnp.int32)], axis=1)
    W_in_p = jnp.pad(W_in, ((0, 0), (0, DIMP - DIM)))
    inrows = _sc_inrows(inword, W_in_p)
    scores = _sc_scores(inrows, owp, negword, W_out)
    loss2d = pl.pallas_call(
        _tc_loss_body,
        grid=(TC_GRID,),
        in_specs=[pl.BlockSpec((TC_BLK, TC_COLS), lambda i: (i, 0))],
        out_specs=pl.BlockSpec(
            (1, 1), lambda i: (0, 0), memory_space=pltpu.SMEM),
        out_shape=jax.ShapeDtypeStruct((1, 1), jnp.float32),
    )(scores.reshape(TC_ROWS, TC_COLS))
    return loss2d[0, 0]


# speedup vs baseline: 1.0490x; 1.0490x over previous
import functools

import jax
import jax.numpy as jnp
from jax import lax
from jax.experimental import pallas as pl
from jax.experimental.pallas import tpu as pltpu
from jax.experimental.pallas import tpu_sc as plsc

VOCAB = 1_000_000
DIM = 64
DIMP = 128
BATCH = 4096
CTX = 20
NEG = 10

NPOS = CTX
NPOSP = 24
NNEG = CTX * NEG
ROWS_B = NPOSP + NNEG
LANES = 16
GROUPS_B = ROWS_B // LANES

NWORKERS = 32
B_PER_W = BATCH // NWORKERS
NBUF = 4
CHUNK_ROWS = ROWS_B
N_CHUNKS = B_PER_W
GATHERS = 2
GLEN = CHUNK_ROWS // GATHERS
TOTAL_ROWS = BATCH * ROWS_B
TC_COLS = 128
TC_ROWS = TOTAL_ROWS // TC_COLS
TC_GRID = 8
TC_BLK = TC_ROWS // TC_GRID


def _sc_body(inword_hbm, owp_hbm, nw_hbm, win_hbm, wout_hbm, out_hbm,
             inidx_v, inrows_v, cidx_v, rows_v, sc_v, *sems):
    nc = 2
    wid = lax.axis_index("s") * nc + lax.axis_index("c")
    sem_row = sems[0:NBUF]
    sem_idx = sems[NBUF:2 * NBUF]
    sem_sc = sems[2 * NBUF:3 * NBUF]

    pltpu.sync_copy(inword_hbm.at[pl.ds(wid * B_PER_W, B_PER_W)], inidx_v)
    pltpu.async_copy(win_hbm.at[inidx_v], inrows_v, sem_row[0]).wait()

    iota = lax.iota(jnp.int32, LANES)
    base_b = wid * B_PER_W

    def fire_idx(chunk, buf):
        b = base_b + chunk
        pltpu.async_copy(
            owp_hbm.at[b], cidx_v.at[buf, pl.ds(0, NPOSP)], sem_idx[buf])
        pltpu.async_copy(
            nw_hbm.at[b], cidx_v.at[buf, pl.ds(NPOSP, NNEG)], sem_idx[buf])

    def drain_idx(buf):
        pltpu.make_async_copy(
            owp_hbm.at[0], cidx_v.at[buf, pl.ds(0, NPOSP)],
            sem_idx[buf]).wait()
        pltpu.make_async_copy(
            nw_hbm.at[0], cidx_v.at[buf, pl.ds(NPOSP, NNEG)],
            sem_idx[buf]).wait()

    def fire_rows(buf):
        for j in range(GATHERS):
            pltpu.async_copy(
                wout_hbm.at[cidx_v.at[buf, pl.ds(j * GLEN, GLEN)]],
                rows_v.at[buf, pl.ds(j * GLEN, GLEN)],
                sem_row[buf])

    def drain_rows(buf):
        for j in range(GATHERS):
            pltpu.make_async_copy(
                wout_hbm.at[pl.ds(0, GLEN)],
                rows_v.at[buf, pl.ds(j * GLEN, GLEN)],
                sem_row[buf]).wait()

    for b in range(NBUF):
        fire_idx(b, b)
    for b in range(NBUF - 1):
        drain_idx(b)
        fire_rows(b)

    def quad_body(cc, carry):
        for par in range(NBUF):
            chunk = cc * NBUF + par
            buf = par

            drain_rows(buf)

            @pl.when(chunk + NBUF - 1 < N_CHUNKS)
            def _fire():
                drain_idx((buf + NBUF - 1) % NBUF)
                fire_rows((buf + NBUF - 1) % NBUF)

            @pl.when(chunk + NBUF < N_CHUNKS)
            def _pref():
                fire_idx(chunk + NBUF, buf)

            @pl.when(chunk >= NBUF)
            def _drain_sc():
                pltpu.make_async_copy(
                    out_hbm.at[pl.ds(0, CHUNK_ROWS)], sc_v.at[buf],
                    sem_sc[buf]).wait()

            b_idx = chunk
            wv = [inrows_v[b_idx, pl.ds(q * LANES, LANES)]
                  for q in range(DIM // LANES)]

            def group_body(g, gc, wv=wv, buf=buf):
                base_row = g * LANES
                res = jnp.zeros((LANES,), jnp.float32)
                for r in range(LANES):
                    row = base_row + r
                    p = ((rows_v[buf, row, pl.ds(0, LANES)] * wv[0]
                          + rows_v[buf, row, pl.ds(LANES, LANES)] * wv[1])
                         + (rows_v[buf, row, pl.ds(2 * LANES, LANES)] * wv[2]
                            + rows_v[buf, row, pl.ds(3 * LANES, LANES)]
                            * wv[3]))
                    s = jnp.sum(p)
                    res = jnp.where(iota == r, s, res)
                sc_v[buf, pl.ds(base_row, LANES)] = res
                return gc

            lax.fori_loop(0, GROUPS_B, group_body, 0)

            gchunk = wid * N_CHUNKS + chunk
            pltpu.async_copy(
                sc_v.at[buf],
                out_hbm.at[pl.ds(gchunk * CHUNK_ROWS, CHUNK_ROWS)],
                sem_sc[buf])
        return carry

    lax.fori_loop(0, N_CHUNKS // NBUF, quad_body, 0)
    for b in range(NBUF):
        pltpu.make_async_copy(
            out_hbm.at[pl.ds(0, CHUNK_ROWS)], sc_v.at[b],
            sem_sc[b]).wait()


def _sc_scores(inword, owp, negword, W_in_p, W_out):
    mesh = plsc.VectorSubcoreMesh(core_axis_name="c", subcore_axis_name="s")
    k = functools.partial(
        pl.kernel,
        mesh=mesh,
        out_type=jax.ShapeDtypeStruct((TOTAL_ROWS,), jnp.float32),
        compiler_params=pltpu.CompilerParams(
            needs_layout_passes=False, use_tc_tiling_on_sc=False),
        scratch_types=[
            pltpu.VMEM((B_PER_W,), jnp.int32),
            pltpu.VMEM((B_PER_W, DIMP), jnp.float32),
            pltpu.VMEM((NBUF, CHUNK_ROWS), jnp.int32),
            pltpu.VMEM((NBUF, CHUNK_ROWS, DIM), jnp.float32),
            pltpu.VMEM((NBUF, CHUNK_ROWS), jnp.float32),
        ] + [pltpu.SemaphoreType.DMA] * (3 * NBUF),
    )(_sc_body)
    return k(inword, owp, negword, W_in_p, W_out)


def _tc_loss_body(s_ref, o_ref):
    pid = pl.program_id(0)
    x = s_ref[...]
    r = lax.broadcasted_iota(jnp.int32, (TC_BLK, TC_COLS), 0)
    c = lax.broadcasted_iota(jnp.int32, (TC_BLK, TC_COLS), 1)
    flat = (pid * TC_BLK + r) * TC_COLS + c
    j = flat % ROWS_B
    z = jnp.where(j < NPOS, x, -x)
    ls = jnp.minimum(z, 0.0) - jnp.log(1.0 + jnp.exp(-jnp.abs(z)))
    pad = jnp.logical_and(j >= NPOS, j < NPOSP)
    val = jnp.where(pad, 0.0, ls)

    @pl.when(pid == 0)
    def _init():
        o_ref[0, 0] = 0.0

    o_ref[0, 0] += jnp.sum(val)

    @pl.when(pid == TC_GRID - 1)
    def _fini():
        o_ref[0, 0] = o_ref[0, 0] * (-1.0 / (BATCH * CTX))


def kernel(inword, outword, negword, W_in, W_out):
    owp = jnp.concatenate(
        [outword, jnp.zeros((BATCH, NPOSP - NPOS), jnp.int32)], axis=1)
    W_in_p = jnp.pad(W_in, ((0, 0), (0, DIMP - DIM)))
    scores = _sc_scores(inword, owp, negword, W_in_p, W_out)
    loss2d = pl.pallas_call(
        _tc_loss_body,
        grid=(TC_GRID,),
        in_specs=[pl.BlockSpec((TC_BLK, TC_COLS), lambda i: (i, 0))],
        out_specs=pl.BlockSpec(
            (1, 1), lambda i: (0, 0), memory_space=pltpu.SMEM),
        out_shape=jax.ShapeDtypeStruct((1, 1), jnp.float32),
    )(scores.reshape(TC_ROWS, TC_COLS))
    return loss2d[0, 0]
